# baseline (speedup 1.0000x reference)
import functools

import jax
import jax.numpy as jnp
from jax import lax
from jax.experimental import pallas as pl
from jax.experimental.pallas import tpu as pltpu
from jax.experimental.pallas import tpu_sc as plsc

F32 = jnp.float32
D = 768
H = 12
DH = 64
DFF = 3072
E = 8
DEPTH = 4
PP = 16
NPATCH = 14
T = NPATCH * NPATCH + 1
NB = 8
NTOK = NB * T
NCLS = 1000
C = int(1.25 * NTOK * 2 / E)
NSLOT = E * C
XE_ROWS = NSLOT + C
NW = 32
TPAD = 1792
CHUNK = TPAD // NW
TB = TPAD // NB


def _ln(x, s, b):
    m = jnp.mean(x, axis=-1, keepdims=True)
    v = jnp.mean((x - m) ** 2, axis=-1, keepdims=True)
    return (x - m) / jnp.sqrt(v + 1e-6) * s + b


def _attn_body(x, ls, lb, wqkv, bqkv, wp, bp):
    xn = _ln(x, ls, lb)
    qkv = jnp.dot(xn, wqkv, preferred_element_type=F32) + bqkv
    outs = []
    for hh in range(H):
        q = qkv[:, hh * DH:(hh + 1) * DH]
        k = qkv[:, D + hh * DH:D + (hh + 1) * DH]
        v = qkv[:, 2 * D + hh * DH:2 * D + (hh + 1) * DH]
        s = lax.dot_general(q, k, (((1,), (1,)), ((), ())),
                            preferred_element_type=F32) * (DH ** -0.5)
        a = jax.nn.softmax(s, axis=-1)
        outs.append(jnp.dot(a, v, preferred_element_type=F32))
    o = jnp.concatenate(outs, axis=1)
    return x + jnp.dot(o, wp, preferred_element_type=F32) + bp


def _mlp_body(x, ls, lb, w1, b1, w2, b2):
    z = _ln(x, ls, lb)
    f = jax.nn.gelu(jnp.dot(z, w1, preferred_element_type=F32) + b1)
    return x + jnp.dot(f, w2, preferred_element_type=F32) + b2


def _attn_kern(h_ref, ls_ref, lb_ref, wqkv_ref, bqkv_ref, wp_ref, bp_ref, o_ref):
    o_ref[0] = _attn_body(h_ref[0], ls_ref[...], lb_ref[...], wqkv_ref[...],
                          bqkv_ref[...], wp_ref[...], bp_ref[...])


_ATTN_W_SPECS = [
    pl.BlockSpec((1, D), lambda b: (0, 0)),
    pl.BlockSpec((1, D), lambda b: (0, 0)),
    pl.BlockSpec((D, 3 * H * DH), lambda b: (0, 0)),
    pl.BlockSpec((1, 3 * H * DH), lambda b: (0, 0)),
    pl.BlockSpec((H * DH, D), lambda b: (0, 0)),
    pl.BlockSpec((1, D), lambda b: (0, 0)),
]


def _attn(h, ls, lb, wqkv, bqkv, wp, bp):
    return pl.pallas_call(
        _attn_kern,
        grid=(NB,),
        in_specs=[pl.BlockSpec((1, T, D), lambda b: (b, 0, 0))] + _ATTN_W_SPECS,
        out_specs=pl.BlockSpec((1, T, D), lambda b: (b, 0, 0)),
        out_shape=jax.ShapeDtypeStruct((NB, T, D), F32),
    )(h, ls.reshape(1, D), lb.reshape(1, D), wqkv, bqkv.reshape(1, -1), wp,
      bp.reshape(1, D))


def _attn_args(ls, lb, wqkv, bqkv, wp, bp):
    return (ls.reshape(1, D), lb.reshape(1, D), wqkv, bqkv.reshape(1, -1), wp,
            bp.reshape(1, D))


_MLP_W_SPECS = [
    pl.BlockSpec((1, D), lambda b: (0, 0)),
    pl.BlockSpec((1, D), lambda b: (0, 0)),
    pl.BlockSpec((D, DFF), lambda b: (0, 0)),
    pl.BlockSpec((1, DFF), lambda b: (0, 0)),
    pl.BlockSpec((DFF, D), lambda b: (0, 0)),
    pl.BlockSpec((1, D), lambda b: (0, 0)),
]


def _mlp_args(ls, lb, w1, b1, w2, b2):
    return (ls.reshape(1, D), lb.reshape(1, D), w1, b1.reshape(1, DFF), w2,
            b2.reshape(1, D))


def _embed_attn_mlp_kern(p_ref, w_ref, b_ref, cls_ref, pos_ref,
                         ls1_ref, lb1_ref, wqkv_ref, bqkv_ref, wp_ref, bp_ref,
                         ls2_ref, lb2_ref, w1_ref, b1_ref, w2_ref, b2_ref,
                         o_ref):
    mm = jnp.dot(p_ref[0], w_ref[...], preferred_element_type=F32) + b_ref[...]
    h0 = jnp.concatenate([cls_ref[0], mm], axis=0) + pos_ref[0]
    h1 = _attn_body(h0, ls1_ref[...], lb1_ref[...], wqkv_ref[...],
                    bqkv_ref[...], wp_ref[...], bp_ref[...])
    o_ref[0] = _mlp_body(h1, ls2_ref[...], lb2_ref[...], w1_ref[...],
                         b1_ref[...], w2_ref[...], b2_ref[...])


def _embed_attn_mlp(p, patch_w, patch_b, cls_tok, pos, attn_args, mlp_args):
    return pl.pallas_call(
        _embed_attn_mlp_kern,
        grid=(NB,),
        in_specs=[
            pl.BlockSpec((1, NPATCH * NPATCH, 3 * PP * PP), lambda b: (b, 0, 0)),
            pl.BlockSpec((3 * PP * PP, D), lambda b: (0, 0)),
            pl.BlockSpec((1, D), lambda b: (0, 0)),
            pl.BlockSpec((1, 1, D), lambda b: (0, 0, 0)),
            pl.BlockSpec((1, T, D), lambda b: (0, 0, 0)),
        ] + _ATTN_W_SPECS + _MLP_W_SPECS,
        out_specs=pl.BlockSpec((1, T, D), lambda b: (b, 0, 0)),
        out_shape=jax.ShapeDtypeStruct((NB, T, D), F32),
    )(p, patch_w, patch_b.reshape(1, D), cls_tok, pos, *attn_args, *mlp_args)


def _attn_comb_mlp_kern(h_ref, g1_ref, g2_ref, o1_ref, o2_ref,
                        ls1_ref, lb1_ref, wqkv_ref, bqkv_ref, wp_ref, bp_ref,
                        ls2_ref, lb2_ref, w1_ref, b1_ref, w2_ref, b2_ref,
                        o_ref):
    x = (h_ref[0] + g1_ref[0, 0:T] * o1_ref[0, 0:T, :]
         + g2_ref[0, 0:T] * o2_ref[0, 0:T, :])
    h1 = _attn_body(x, ls1_ref[...], lb1_ref[...], wqkv_ref[...],
                    bqkv_ref[...], wp_ref[...], bp_ref[...])
    o_ref[0] = _mlp_body(h1, ls2_ref[...], lb2_ref[...], w1_ref[...],
                         b1_ref[...], w2_ref[...], b2_ref[...])


def _attn_comb_mlp(h, g1, g2, o1, o2, attn_args, mlp_args):
    return pl.pallas_call(
        _attn_comb_mlp_kern,
        grid=(NB,),
        in_specs=[
            pl.BlockSpec((1, T, D), lambda b: (b, 0, 0)),
            pl.BlockSpec((1, TB, 1), lambda b: (b, 0, 0)),
            pl.BlockSpec((1, TB, 1), lambda b: (b, 0, 0)),
            pl.BlockSpec((1, TB, D), lambda b: (b, 0, 0)),
            pl.BlockSpec((1, TB, D), lambda b: (b, 0, 0)),
        ] + _ATTN_W_SPECS + _MLP_W_SPECS,
        out_specs=pl.BlockSpec((1, T, D), lambda b: (b, 0, 0)),
        out_shape=jax.ShapeDtypeStruct((NB, T, D), F32),
    )(h, g1.reshape(NB, TB, 1), g2.reshape(NB, TB, 1),
      o1.reshape(NB, TB, D), o2.reshape(NB, TB, D), *attn_args, *mlp_args)


def _route_kern(x_ref, ls_ref, lb_ref, wg_ref, z_ref,
                s1d_ref, s2d_ref, s1c_ref, s2c_ref, g1_ref, g2_ref, ne_ref):
    x = x_ref[...]
    z = _ln(x, ls_ref[...], lb_ref[...])
    z_ref[0:NTOK] = z
    z_ref[NTOK:TPAD] = jnp.zeros((TPAD - NTOK, D), F32)
    logits = jnp.dot(z, wg_ref[...], preferred_element_type=F32)
    gates = jax.nn.softmax(logits, axis=-1)
    eio = lax.broadcasted_iota(jnp.int32, (NTOK, E), 1)
    v1 = jnp.max(gates, axis=-1, keepdims=True)
    i1 = jnp.min(jnp.where(gates >= v1, eio, E), axis=-1, keepdims=True)
    m1 = (eio == i1).astype(F32)
    gates2 = gates - m1 * 2.0
    v2 = jnp.max(gates2, axis=-1, keepdims=True)
    i2 = jnp.min(jnp.where(gates2 >= v2, eio, E), axis=-1, keepdims=True)
    m2 = (eio == i2).astype(F32)
    rio = lax.broadcasted_iota(jnp.int32, (NTOK, NTOK), 0)
    cio = lax.broadcasted_iota(jnp.int32, (NTOK, NTOK), 1)
    ltri = (rio >= cio).astype(F32)
    loc1 = jnp.dot(ltri, m1, preferred_element_type=F32) - 1.0
    cnt1 = jnp.sum(m1, axis=0, keepdims=True)
    loc2 = jnp.dot(ltri, m2, preferred_element_type=F32) - 1.0 + cnt1
    m1k = m1 * (loc1 < C).astype(F32)
    m2k = m2 * (loc2 < C).astype(F32)
    p1 = jnp.sum(loc1 * m1k, axis=-1, keepdims=True)
    p2 = jnp.sum(loc2 * m2k, axis=-1, keepdims=True)
    k1 = jnp.sum(m1k, axis=-1, keepdims=True)
    k2 = jnp.sum(m2k, axis=-1, keepdims=True)
    den = v1 + v2 + 1e-9
    ne_ref[...] = jnp.sum(m1k + m2k, axis=0, keepdims=True)
    slot1 = i1 * C + p1.astype(jnp.int32)
    slot2 = i2 * C + p2.astype(jnp.int32)
    kept1 = k1 > 0.0
    kept2 = k2 > 0.0
    pad = jnp.full((TPAD - NTOK, 1), NSLOT, jnp.int32)
    s1d_ref[0:NTOK] = jnp.where(kept1, slot1, NSLOT)
    s1d_ref[NTOK:TPAD] = pad
    s2d_ref[0:NTOK] = jnp.where(kept2, slot2, NSLOT)
    s2d_ref[NTOK:TPAD] = pad
    s1c = jnp.where(kept1, slot1, 0)
    s2c = jnp.where(kept2, slot2, 0)
    s1c_ref[...] = jnp.zeros((TPAD, 1), jnp.int32)
    s2c_ref[...] = jnp.zeros((TPAD, 1), jnp.int32)
    g1_ref[...] = jnp.zeros((TPAD, 1), F32)
    g2_ref[...] = jnp.zeros((TPAD, 1), F32)
    gv1 = v1 / den * k1
    gv2 = v2 / den * k2
    for b in range(NB):
        s1c_ref[b * TB:b * TB + T] = s1c[b * T:(b + 1) * T]
        s2c_ref[b * TB:b * TB + T] = s2c[b * T:(b + 1) * T]
        g1_ref[b * TB:b * TB + T] = gv1[b * T:(b + 1) * T]
        g2_ref[b * TB:b * TB + T] = gv2[b * T:(b + 1) * T]


def _route(xflat, ls, lb, wg):
    icol = jax.ShapeDtypeStruct((TPAD, 1), jnp.int32)
    col = jax.ShapeDtypeStruct((TPAD, 1), F32)
    return pl.pallas_call(
        _route_kern,
        in_specs=[
            pl.BlockSpec((NTOK, D), lambda: (0, 0)),
            pl.BlockSpec((1, D), lambda: (0, 0)),
            pl.BlockSpec((1, D), lambda: (0, 0)),
            pl.BlockSpec((D, E), lambda: (0, 0)),
        ],
        out_specs=[pl.BlockSpec((TPAD, D), lambda: (0, 0))]
        + [pl.BlockSpec((TPAD, 1), lambda: (0, 0))] * 6
        + [pl.BlockSpec((1, E), lambda: (0, 0))],
        out_shape=[jax.ShapeDtypeStruct((TPAD, D), F32)]
        + [icol] * 4 + [col] * 2
        + [jax.ShapeDtypeStruct((1, E), F32)],
    )(xflat, ls.reshape(1, D), lb.reshape(1, D), wg)


@functools.cache
def _make_sc_kernels():
    mesh = plsc.VectorSubcoreMesh(core_axis_name="c", subcore_axis_name="s")

    @functools.partial(
        pl.kernel, mesh=mesh,
        out_type=jax.ShapeDtypeStruct((XE_ROWS, D), F32),
        scratch_types=[
            pltpu.VMEM((CHUNK,), jnp.int32),
            pltpu.VMEM((CHUNK,), jnp.int32),
            pltpu.VMEM((CHUNK, D), F32),
            pltpu.SemaphoreType.DMA,
            pltpu.SemaphoreType.DMA,
            pltpu.SemaphoreType.DMA,
        ],
    )
    def sc_dispatch(z_hbm, s1_hbm, s2_hbm, out_hbm,
                    idx1_v, idx2_v, rows_v, sem_r, sem1, sem2):
        wid = lax.axis_index("s") * 2 + lax.axis_index("c")
        base = wid * CHUNK
        cz = pltpu.async_copy(z_hbm.at[pl.ds(base, CHUNK)], rows_v, sem_r)
        pltpu.sync_copy(s1_hbm.at[pl.ds(base, CHUNK)], idx1_v)
        pltpu.sync_copy(s2_hbm.at[pl.ds(base, CHUNK)], idx2_v)
        cz.wait()
        c1 = pltpu.async_copy(rows_v, out_hbm.at[idx1_v], sem1)
        c2 = pltpu.async_copy(rows_v, out_hbm.at[idx2_v], sem2)
        c1.wait()
        c2.wait()

    @functools.partial(
        pl.kernel, mesh=mesh,
        out_type=(jax.ShapeDtypeStruct((TPAD, D), F32),
                  jax.ShapeDtypeStruct((TPAD, D), F32)),
        scratch_types=[
            pltpu.VMEM((CHUNK,), jnp.int32),
            pltpu.VMEM((CHUNK,), jnp.int32),
            pltpu.VMEM((CHUNK, D), F32),
            pltpu.VMEM((CHUNK, D), F32),
            pltpu.SemaphoreType.DMA,
            pltpu.SemaphoreType.DMA,
            pltpu.SemaphoreType.DMA,
            pltpu.SemaphoreType.DMA,
        ],
    )
    def sc_gather(o_hbm, s1_hbm, s2_hbm, o1_hbm, o2_hbm,
                  idx1_v, idx2_v, rows1_v, rows2_v, sem1, sem2, sem3, sem4):
        wid = lax.axis_index("s") * 2 + lax.axis_index("c")
        base = wid * CHUNK
        pltpu.sync_copy(s1_hbm.at[pl.ds(base, CHUNK)], idx1_v)
        pltpu.sync_copy(s2_hbm.at[pl.ds(base, CHUNK)], idx2_v)
        c1 = pltpu.async_copy(o_hbm.at[idx1_v], rows1_v, sem1)
        c2 = pltpu.async_copy(o_hbm.at[idx2_v], rows2_v, sem2)
        c1.wait()
        c3 = pltpu.async_copy(rows1_v, o1_hbm.at[pl.ds(base, CHUNK)], sem3)
        c2.wait()
        c4 = pltpu.async_copy(rows2_v, o2_hbm.at[pl.ds(base, CHUNK)], sem4)
        c3.wait()
        c4.wait()

    return sc_dispatch, sc_gather


def _sc_dispatch(z, s1, s2):
    return _make_sc_kernels()[0](z, s1, s2)


def _sc_gather(o, s1, s2):
    return _make_sc_kernels()[1](o, s1, s2)


def _expert_kern(x_ref, ne_ref, w1_ref, b1_ref, w2_ref, b2_ref, o_ref):
    rio = lax.broadcasted_iota(jnp.int32, (C, 1), 0)
    mask = (rio < ne_ref[0].astype(jnp.int32)).astype(F32)
    x = x_ref[0] * mask
    hmid = jax.nn.gelu(jnp.dot(x, w1_ref[0], preferred_element_type=F32)
                       + b1_ref[0])
    o_ref[0] = jnp.dot(hmid, w2_ref[0], preferred_element_type=F32) + b2_ref[0]


def _experts(xe, ne, w1, b1, w2, b2):
    return pl.pallas_call(
        _expert_kern,
        grid=(E,),
        in_specs=[
            pl.BlockSpec((1, C, D), lambda e: (e, 0, 0)),
            pl.BlockSpec((1, 1, 1), lambda e: (e, 0, 0)),
            pl.BlockSpec((1, D, DFF), lambda e: (e, 0, 0)),
            pl.BlockSpec((1, 1, DFF), lambda e: (e, 0, 0)),
            pl.BlockSpec((1, DFF, D), lambda e: (e, 0, 0)),
            pl.BlockSpec((1, 1, D), lambda e: (e, 0, 0)),
        ],
        out_specs=pl.BlockSpec((1, C, D), lambda e: (e, 0, 0)),
        out_shape=jax.ShapeDtypeStruct((E, C, D), F32),
    )(xe, ne.reshape(E, 1, 1), w1, b1.reshape(E, 1, DFF), w2,
      b2.reshape(E, 1, D))


def _final_kern(h_ref, g1_ref, g2_ref, o1_ref, o2_ref, ls_ref, lb_ref,
                w_ref, b_ref, o_ref):
    x = (h_ref[...] + g1_ref[:, 0:T] * o1_ref[:, 0:T, :]
         + g2_ref[:, 0:T] * o2_ref[:, 0:T, :])
    xn = _ln(x, ls_ref[...], lb_ref[...])
    m = jnp.mean(xn, axis=1)
    o_ref[...] = jnp.dot(m, w_ref[...], preferred_element_type=F32) + b_ref[...]


def _final(h, g1, g2, o1, o2, ls, lb, w, b):
    return pl.pallas_call(
        _final_kern,
        in_specs=[
            pl.BlockSpec((NB, T, D), lambda: (0, 0, 0)),
            pl.BlockSpec((NB, TB, 1), lambda: (0, 0, 0)),
            pl.BlockSpec((NB, TB, 1), lambda: (0, 0, 0)),
            pl.BlockSpec((NB, TB, D), lambda: (0, 0, 0)),
            pl.BlockSpec((NB, TB, D), lambda: (0, 0, 0)),
            pl.BlockSpec((1, D), lambda: (0, 0)),
            pl.BlockSpec((1, D), lambda: (0, 0)),
            pl.BlockSpec((D, NCLS), lambda: (0, 0)),
            pl.BlockSpec((1, NCLS), lambda: (0, 0)),
        ],
        out_specs=pl.BlockSpec((NB, NCLS), lambda: (0, 0)),
        out_shape=jax.ShapeDtypeStruct((NB, NCLS), F32),
    )(h, g1.reshape(NB, TB, 1), g2.reshape(NB, TB, 1),
      o1.reshape(NB, TB, D), o2.reshape(NB, TB, D),
      ls.reshape(1, D), lb.reshape(1, D), w, b.reshape(1, NCLS))


def kernel(x, patch_w, patch_b, cls_tok, pos, ln1_s, ln1_b, qkv_w, qkv_b,
           proj_w, proj_b, ln2_s, ln2_b, mlp_w1, mlp_b1, mlp_w2, mlp_b2,
           gate_w, moe_w1, moe_b1, moe_w2, moe_b2, lnf_s, lnf_b, cls_w, cls_b):
    p = x.reshape(NB, 3, NPATCH, PP, NPATCH, PP)
    p = p.transpose(0, 2, 4, 1, 3, 5).reshape(NB, NPATCH * NPATCH, 3 * PP * PP)

    def moe(h, i):
        j = i // 2
        (z, s1d, s2d, s1c, s2c, g1, g2, ne) = _route(
            h.reshape(NTOK, D), ln2_s[i], ln2_b[i], gate_w[j])
        xe = _sc_dispatch(z, s1d.reshape(TPAD), s2d.reshape(TPAD))
        o = _experts(xe.reshape(XE_ROWS // C, C, D), ne,
                     moe_w1[j], moe_b1[j], moe_w2[j], moe_b2[j])
        o1, o2 = _sc_gather(o.reshape(NSLOT, D), s1c.reshape(TPAD),
                            s2c.reshape(TPAD))
        return g1, g2, o1, o2

    h = _embed_attn_mlp(
        p, patch_w, patch_b, cls_tok, pos,
        _attn_args(ln1_s[0], ln1_b[0], qkv_w[0], qkv_b[0], proj_w[0],
                   proj_b[0]),
        _mlp_args(ln2_s[0], ln2_b[0], mlp_w1[0], mlp_b1[0], mlp_w2[0],
                  mlp_b2[0]))
    h = _attn(h, ln1_s[1], ln1_b[1], qkv_w[1], qkv_b[1], proj_w[1], proj_b[1])
    g1, g2, o1, o2 = moe(h, 1)
    h = _attn_comb_mlp(
        h, g1, g2, o1, o2,
        _attn_args(ln1_s[2], ln1_b[2], qkv_w[2], qkv_b[2], proj_w[2],
                   proj_b[2]),
        _mlp_args(ln2_s[2], ln2_b[2], mlp_w1[1], mlp_b1[1], mlp_w2[1],
                  mlp_b2[1]))
    h = _attn(h, ln1_s[3], ln1_b[3], qkv_w[3], qkv_b[3], proj_w[3], proj_b[3])
    g1, g2, o1, o2 = moe(h, 3)
    return _final(h, g1, g2, o1, o2, lnf_s, lnf_b, cls_w, cls_b)


# speedup vs baseline: 1.0771x; 1.0771x over previous
import functools

import jax
import jax.numpy as jnp
from jax import lax
from jax.experimental import pallas as pl
from jax.experimental.pallas import tpu as pltpu
from jax.experimental.pallas import tpu_sc as plsc

F32 = jnp.float32
D = 768
H = 12
DH = 64
DFF = 3072
E = 8
DEPTH = 4
PP = 16
NPATCH = 14
T = NPATCH * NPATCH + 1
NB = 8
NTOK = NB * T
NCLS = 1000
C = int(1.25 * NTOK * 2 / E)
NSLOT = E * C
XE_ROWS = NSLOT + C
NW = 32
TPAD = 1792
CHUNK = TPAD // NW
TB = TPAD // NB


def _ln(x, s, b):
    m = jnp.mean(x, axis=-1, keepdims=True)
    v = jnp.mean((x - m) ** 2, axis=-1, keepdims=True)
    return (x - m) / jnp.sqrt(v + 1e-6) * s + b


def _attn_body(x, ls, lb, wqkv, bqkv, wp, bp):
    xn = _ln(x, ls, lb)
    qkv = jnp.dot(xn, wqkv, preferred_element_type=F32) + bqkv
    outs = []
    for hh in range(H):
        q = qkv[:, hh * DH:(hh + 1) * DH]
        k = qkv[:, D + hh * DH:D + (hh + 1) * DH]
        v = qkv[:, 2 * D + hh * DH:2 * D + (hh + 1) * DH]
        s = lax.dot_general(q, k, (((1,), (1,)), ((), ())),
                            preferred_element_type=F32) * (DH ** -0.5)
        a = jax.nn.softmax(s, axis=-1)
        outs.append(jnp.dot(a, v, preferred_element_type=F32))
    o = jnp.concatenate(outs, axis=1)
    return x + jnp.dot(o, wp, preferred_element_type=F32) + bp


def _mlp_body(x, ls, lb, w1, b1, w2, b2):
    z = _ln(x, ls, lb)
    f = jax.nn.gelu(jnp.dot(z, w1, preferred_element_type=F32) + b1)
    return x + jnp.dot(f, w2, preferred_element_type=F32) + b2


def _attn_kern(h_ref, ls_ref, lb_ref, wqkv_ref, bqkv_ref, wp_ref, bp_ref, o_ref):
    o_ref[0] = _attn_body(h_ref[0], ls_ref[...], lb_ref[...], wqkv_ref[...],
                          bqkv_ref[...], wp_ref[...], bp_ref[...])


_ATTN_W_SPECS = [
    pl.BlockSpec((1, D), lambda b: (0, 0)),
    pl.BlockSpec((1, D), lambda b: (0, 0)),
    pl.BlockSpec((D, 3 * H * DH), lambda b: (0, 0)),
    pl.BlockSpec((1, 3 * H * DH), lambda b: (0, 0)),
    pl.BlockSpec((H * DH, D), lambda b: (0, 0)),
    pl.BlockSpec((1, D), lambda b: (0, 0)),
]


def _attn(h, ls, lb, wqkv, bqkv, wp, bp):
    return pl.pallas_call(
        _attn_kern,
        grid=(NB,),
        in_specs=[pl.BlockSpec((1, T, D), lambda b: (b, 0, 0))] + _ATTN_W_SPECS,
        out_specs=pl.BlockSpec((1, T, D), lambda b: (b, 0, 0)),
        out_shape=jax.ShapeDtypeStruct((NB, T, D), F32),
    )(h, ls.reshape(1, D), lb.reshape(1, D), wqkv, bqkv.reshape(1, -1), wp,
      bp.reshape(1, D))


def _attn_args(ls, lb, wqkv, bqkv, wp, bp):
    return (ls.reshape(1, D), lb.reshape(1, D), wqkv, bqkv.reshape(1, -1), wp,
            bp.reshape(1, D))


_MLP_W_SPECS = [
    pl.BlockSpec((1, D), lambda b: (0, 0)),
    pl.BlockSpec((1, D), lambda b: (0, 0)),
    pl.BlockSpec((D, DFF), lambda b: (0, 0)),
    pl.BlockSpec((1, DFF), lambda b: (0, 0)),
    pl.BlockSpec((DFF, D), lambda b: (0, 0)),
    pl.BlockSpec((1, D), lambda b: (0, 0)),
]


def _mlp_args(ls, lb, w1, b1, w2, b2):
    return (ls.reshape(1, D), lb.reshape(1, D), w1, b1.reshape(1, DFF), w2,
            b2.reshape(1, D))


def _embed_attn_mlp_kern(p_ref, w_ref, b_ref, cls_ref, pos_ref,
                         ls1_ref, lb1_ref, wqkv_ref, bqkv_ref, wp_ref, bp_ref,
                         ls2_ref, lb2_ref, w1_ref, b1_ref, w2_ref, b2_ref,
                         o_ref):
    mm = jnp.dot(p_ref[0], w_ref[...], preferred_element_type=F32) + b_ref[...]
    h0 = jnp.concatenate([cls_ref[0], mm], axis=0) + pos_ref[0]
    h1 = _attn_body(h0, ls1_ref[...], lb1_ref[...], wqkv_ref[...],
                    bqkv_ref[...], wp_ref[...], bp_ref[...])
    o_ref[0] = _mlp_body(h1, ls2_ref[...], lb2_ref[...], w1_ref[...],
                         b1_ref[...], w2_ref[...], b2_ref[...])


def _embed_attn_mlp(p, patch_w, patch_b, cls_tok, pos, attn_args, mlp_args):
    return pl.pallas_call(
        _embed_attn_mlp_kern,
        grid=(NB,),
        in_specs=[
            pl.BlockSpec((1, NPATCH * NPATCH, 3 * PP * PP), lambda b: (b, 0, 0)),
            pl.BlockSpec((3 * PP * PP, D), lambda b: (0, 0)),
            pl.BlockSpec((1, D), lambda b: (0, 0)),
            pl.BlockSpec((1, 1, D), lambda b: (0, 0, 0)),
            pl.BlockSpec((1, T, D), lambda b: (0, 0, 0)),
        ] + _ATTN_W_SPECS + _MLP_W_SPECS,
        out_specs=pl.BlockSpec((1, T, D), lambda b: (b, 0, 0)),
        out_shape=jax.ShapeDtypeStruct((NB, T, D), F32),
    )(p, patch_w, patch_b.reshape(1, D), cls_tok, pos, *attn_args, *mlp_args)


def _comb_rows(s1, s2, g1, g2, o):
    sio = lax.broadcasted_iota(jnp.int32, (T, NSLOT), 1)
    comb = jnp.where(sio == s1, g1, 0.0) + jnp.where(sio == s2, g2, 0.0)
    return jnp.dot(comb, o, preferred_element_type=F32)


def _attn_comb_mlp_kern(h_ref, s1_ref, s2_ref, g1_ref, g2_ref, o_ref,
                        ls1_ref, lb1_ref, wqkv_ref, bqkv_ref, wp_ref, bp_ref,
                        ls2_ref, lb2_ref, w1_ref, b1_ref, w2_ref, b2_ref,
                        y_ref):
    x = h_ref[0] + _comb_rows(s1_ref[0, 0:T], s2_ref[0, 0:T],
                              g1_ref[0, 0:T], g2_ref[0, 0:T], o_ref[...])
    h1 = _attn_body(x, ls1_ref[...], lb1_ref[...], wqkv_ref[...],
                    bqkv_ref[...], wp_ref[...], bp_ref[...])
    y_ref[0] = _mlp_body(h1, ls2_ref[...], lb2_ref[...], w1_ref[...],
                         b1_ref[...], w2_ref[...], b2_ref[...])


def _attn_comb_mlp(h, s1c, s2c, g1, g2, o, attn_args, mlp_args):
    return pl.pallas_call(
        _attn_comb_mlp_kern,
        grid=(NB,),
        in_specs=[
            pl.BlockSpec((1, T, D), lambda b: (b, 0, 0)),
            pl.BlockSpec((1, TB, 1), lambda b: (b, 0, 0)),
            pl.BlockSpec((1, TB, 1), lambda b: (b, 0, 0)),
            pl.BlockSpec((1, TB, 1), lambda b: (b, 0, 0)),
            pl.BlockSpec((1, TB, 1), lambda b: (b, 0, 0)),
            pl.BlockSpec((NSLOT, D), lambda b: (0, 0)),
        ] + _ATTN_W_SPECS + _MLP_W_SPECS,
        out_specs=pl.BlockSpec((1, T, D), lambda b: (b, 0, 0)),
        out_shape=jax.ShapeDtypeStruct((NB, T, D), F32),
    )(h, s1c.reshape(NB, TB, 1), s2c.reshape(NB, TB, 1),
      g1.reshape(NB, TB, 1), g2.reshape(NB, TB, 1), o, *attn_args, *mlp_args)


def _route_kern(x_ref, ls_ref, lb_ref, wg_ref, z_ref,
                s1d_ref, s2d_ref, s1c_ref, s2c_ref, g1_ref, g2_ref, ne_ref):
    x = x_ref[...]
    z = _ln(x, ls_ref[...], lb_ref[...])
    z_ref[0:NTOK] = z
    z_ref[NTOK:TPAD] = jnp.zeros((TPAD - NTOK, D), F32)
    logits = jnp.dot(z, wg_ref[...], preferred_element_type=F32)
    gates = jax.nn.softmax(logits, axis=-1)
    eio = lax.broadcasted_iota(jnp.int32, (NTOK, E), 1)
    v1 = jnp.max(gates, axis=-1, keepdims=True)
    i1 = jnp.min(jnp.where(gates >= v1, eio, E), axis=-1, keepdims=True)
    m1 = (eio == i1).astype(F32)
    gates2 = gates - m1 * 2.0
    v2 = jnp.max(gates2, axis=-1, keepdims=True)
    i2 = jnp.min(jnp.where(gates2 >= v2, eio, E), axis=-1, keepdims=True)
    m2 = (eio == i2).astype(F32)
    rio = lax.broadcasted_iota(jnp.int32, (NTOK, NTOK), 0)
    cio = lax.broadcasted_iota(jnp.int32, (NTOK, NTOK), 1)
    ltri = (rio >= cio).astype(F32)
    loc1 = jnp.dot(ltri, m1, preferred_element_type=F32) - 1.0
    cnt1 = jnp.sum(m1, axis=0, keepdims=True)
    loc2 = jnp.dot(ltri, m2, preferred_element_type=F32) - 1.0 + cnt1
    m1k = m1 * (loc1 < C).astype(F32)
    m2k = m2 * (loc2 < C).astype(F32)
    p1 = jnp.sum(loc1 * m1k, axis=-1, keepdims=True)
    p2 = jnp.sum(loc2 * m2k, axis=-1, keepdims=True)
    k1 = jnp.sum(m1k, axis=-1, keepdims=True)
    k2 = jnp.sum(m2k, axis=-1, keepdims=True)
    den = v1 + v2 + 1e-9
    ne_ref[...] = jnp.sum(m1k + m2k, axis=0, keepdims=True)
    slot1 = i1 * C + p1.astype(jnp.int32)
    slot2 = i2 * C + p2.astype(jnp.int32)
    kept1 = k1 > 0.0
    kept2 = k2 > 0.0
    pad = jnp.full((TPAD - NTOK, 1), NSLOT, jnp.int32)
    s1d_ref[0:NTOK] = jnp.where(kept1, slot1, NSLOT)
    s1d_ref[NTOK:TPAD] = pad
    s2d_ref[0:NTOK] = jnp.where(kept2, slot2, NSLOT)
    s2d_ref[NTOK:TPAD] = pad
    s1c = jnp.where(kept1, slot1, 0)
    s2c = jnp.where(kept2, slot2, 0)
    s1c_ref[...] = jnp.zeros((TPAD, 1), jnp.int32)
    s2c_ref[...] = jnp.zeros((TPAD, 1), jnp.int32)
    g1_ref[...] = jnp.zeros((TPAD, 1), F32)
    g2_ref[...] = jnp.zeros((TPAD, 1), F32)
    gv1 = v1 / den * k1
    gv2 = v2 / den * k2
    for b in range(NB):
        s1c_ref[b * TB:b * TB + T] = s1c[b * T:(b + 1) * T]
        s2c_ref[b * TB:b * TB + T] = s2c[b * T:(b + 1) * T]
        g1_ref[b * TB:b * TB + T] = gv1[b * T:(b + 1) * T]
        g2_ref[b * TB:b * TB + T] = gv2[b * T:(b + 1) * T]


def _route(xflat, ls, lb, wg):
    icol = jax.ShapeDtypeStruct((TPAD, 1), jnp.int32)
    col = jax.ShapeDtypeStruct((TPAD, 1), F32)
    return pl.pallas_call(
        _route_kern,
        in_specs=[
            pl.BlockSpec((NTOK, D), lambda: (0, 0)),
            pl.BlockSpec((1, D), lambda: (0, 0)),
            pl.BlockSpec((1, D), lambda: (0, 0)),
            pl.BlockSpec((D, E), lambda: (0, 0)),
        ],
        out_specs=[pl.BlockSpec((TPAD, D), lambda: (0, 0))]
        + [pl.BlockSpec((TPAD, 1), lambda: (0, 0))] * 6
        + [pl.BlockSpec((1, E), lambda: (0, 0))],
        out_shape=[jax.ShapeDtypeStruct((TPAD, D), F32)]
        + [icol] * 4 + [col] * 2
        + [jax.ShapeDtypeStruct((1, E), F32)],
    )(xflat, ls.reshape(1, D), lb.reshape(1, D), wg)


@functools.cache
def _make_sc_kernels():
    mesh = plsc.VectorSubcoreMesh(core_axis_name="c", subcore_axis_name="s")

    @functools.partial(
        pl.kernel, mesh=mesh,
        out_type=jax.ShapeDtypeStruct((XE_ROWS, D), F32),
        scratch_types=[
            pltpu.VMEM((CHUNK,), jnp.int32),
            pltpu.VMEM((CHUNK,), jnp.int32),
            pltpu.VMEM((CHUNK, D), F32),
            pltpu.SemaphoreType.DMA,
            pltpu.SemaphoreType.DMA,
            pltpu.SemaphoreType.DMA,
        ],
    )
    def sc_dispatch(z_hbm, s1_hbm, s2_hbm, out_hbm,
                    idx1_v, idx2_v, rows_v, sem_r, sem1, sem2):
        wid = lax.axis_index("s") * 2 + lax.axis_index("c")
        base = wid * CHUNK
        cz = pltpu.async_copy(z_hbm.at[pl.ds(base, CHUNK)], rows_v, sem_r)
        pltpu.sync_copy(s1_hbm.at[pl.ds(base, CHUNK)], idx1_v)
        pltpu.sync_copy(s2_hbm.at[pl.ds(base, CHUNK)], idx2_v)
        cz.wait()
        c1 = pltpu.async_copy(rows_v, out_hbm.at[idx1_v], sem1)
        c2 = pltpu.async_copy(rows_v, out_hbm.at[idx2_v], sem2)
        c1.wait()
        c2.wait()

    return sc_dispatch


def _sc_dispatch(z, s1, s2):
    return _make_sc_kernels()(z, s1, s2)


def _expert_kern(x_ref, ne_ref, w1_ref, b1_ref, w2_ref, b2_ref, o_ref):
    rio = lax.broadcasted_iota(jnp.int32, (C, 1), 0)
    mask = rio < ne_ref[0].astype(jnp.int32)
    x = jnp.where(mask, x_ref[0], 0.0)
    hmid = jax.nn.gelu(jnp.dot(x, w1_ref[0], preferred_element_type=F32)
                       + b1_ref[0])
    o_ref[0] = jnp.dot(hmid, w2_ref[0], preferred_element_type=F32) + b2_ref[0]


def _experts(xe, ne, w1, b1, w2, b2):
    return pl.pallas_call(
        _expert_kern,
        grid=(E,),
        in_specs=[
            pl.BlockSpec((1, C, D), lambda e: (e, 0, 0)),
            pl.BlockSpec((1, 1, 1), lambda e: (e, 0, 0)),
            pl.BlockSpec((1, D, DFF), lambda e: (e, 0, 0)),
            pl.BlockSpec((1, 1, DFF), lambda e: (e, 0, 0)),
            pl.BlockSpec((1, DFF, D), lambda e: (e, 0, 0)),
            pl.BlockSpec((1, 1, D), lambda e: (e, 0, 0)),
        ],
        out_specs=pl.BlockSpec((1, C, D), lambda e: (e, 0, 0)),
        out_shape=jax.ShapeDtypeStruct((E, C, D), F32),
    )(xe, ne.reshape(E, 1, 1), w1, b1.reshape(E, 1, DFF), w2,
      b2.reshape(E, 1, D))


def _final_kern(h_ref, s1_ref, s2_ref, g1_ref, g2_ref, o_ref, ls_ref, lb_ref,
                w_ref, b_ref, y_ref):
    rows = []
    for b in range(NB):
        x = h_ref[b] + _comb_rows(s1_ref[b, 0:T], s2_ref[b, 0:T],
                                  g1_ref[b, 0:T], g2_ref[b, 0:T], o_ref[...])
        xn = _ln(x, ls_ref[...], lb_ref[...])
        rows.append(jnp.mean(xn, axis=0, keepdims=True))
    m = jnp.concatenate(rows, axis=0)
    y_ref[...] = jnp.dot(m, w_ref[...], preferred_element_type=F32) + b_ref[...]


def _final(h, s1c, s2c, g1, g2, o, ls, lb, w, b):
    return pl.pallas_call(
        _final_kern,
        in_specs=[
            pl.BlockSpec((NB, T, D), lambda: (0, 0, 0)),
            pl.BlockSpec((NB, TB, 1), lambda: (0, 0, 0)),
            pl.BlockSpec((NB, TB, 1), lambda: (0, 0, 0)),
            pl.BlockSpec((NB, TB, 1), lambda: (0, 0, 0)),
            pl.BlockSpec((NB, TB, 1), lambda: (0, 0, 0)),
            pl.BlockSpec((NSLOT, D), lambda: (0, 0)),
            pl.BlockSpec((1, D), lambda: (0, 0)),
            pl.BlockSpec((1, D), lambda: (0, 0)),
            pl.BlockSpec((D, NCLS), lambda: (0, 0)),
            pl.BlockSpec((1, NCLS), lambda: (0, 0)),
        ],
        out_specs=pl.BlockSpec((NB, NCLS), lambda: (0, 0)),
        out_shape=jax.ShapeDtypeStruct((NB, NCLS), F32),
    )(h, s1c.reshape(NB, TB, 1), s2c.reshape(NB, TB, 1),
      g1.reshape(NB, TB, 1), g2.reshape(NB, TB, 1), o,
      ls.reshape(1, D), lb.reshape(1, D), w, b.reshape(1, NCLS))


def kernel(x, patch_w, patch_b, cls_tok, pos, ln1_s, ln1_b, qkv_w, qkv_b,
           proj_w, proj_b, ln2_s, ln2_b, mlp_w1, mlp_b1, mlp_w2, mlp_b2,
           gate_w, moe_w1, moe_b1, moe_w2, moe_b2, lnf_s, lnf_b, cls_w, cls_b):
    p = x.reshape(NB, 3, NPATCH, PP, NPATCH, PP)
    p = p.transpose(0, 2, 4, 1, 3, 5).reshape(NB, NPATCH * NPATCH, 3 * PP * PP)

    def moe(h, i):
        j = i // 2
        (z, s1d, s2d, s1c, s2c, g1, g2, ne) = _route(
            h.reshape(NTOK, D), ln2_s[i], ln2_b[i], gate_w[j])
        xe = _sc_dispatch(z, s1d.reshape(TPAD), s2d.reshape(TPAD))
        o = _experts(xe.reshape(XE_ROWS // C, C, D), ne,
                     moe_w1[j], moe_b1[j], moe_w2[j], moe_b2[j])
        return s1c, s2c, g1, g2, o.reshape(NSLOT, D)

    h = _embed_attn_mlp(
        p, patch_w, patch_b, cls_tok, pos,
        _attn_args(ln1_s[0], ln1_b[0], qkv_w[0], qkv_b[0], proj_w[0],
                   proj_b[0]),
        _mlp_args(ln2_s[0], ln2_b[0], mlp_w1[0], mlp_b1[0], mlp_w2[0],
                  mlp_b2[0]))
    h = _attn(h, ln1_s[1], ln1_b[1], qkv_w[1], qkv_b[1], proj_w[1], proj_b[1])
    s1c, s2c, g1, g2, o = moe(h, 1)
    h = _attn_comb_mlp(
        h, s1c, s2c, g1, g2, o,
        _attn_args(ln1_s[2], ln1_b[2], qkv_w[2], qkv_b[2], proj_w[2],
                   proj_b[2]),
        _mlp_args(ln2_s[2], ln2_b[2], mlp_w1[1], mlp_b1[1], mlp_w2[1],
                  mlp_b2[1]))
    h = _attn(h, ln1_s[3], ln1_b[3], qkv_w[3], qkv_b[3], proj_w[3], proj_b[3])
    s1c, s2c, g1, g2, o = moe(h, 3)
    return _final(h, s1c, s2c, g1, g2, o, lnf_s, lnf_b, cls_w, cls_b)
